# Initial kernel scaffold; baseline (speedup 1.0000x reference)
import jax
import jax.numpy as jnp
from jax.experimental import pallas as pl


def kernel(x, adjacency_matrix, W0, a1_0, a2_0, b0):
    raise NotImplementedError("write your pallas kernel here")


# speedup vs baseline: 1.7082x; 1.7082x over previous
import functools

import jax
import jax.numpy as jnp
from jax.experimental import pallas as pl

N = 4096
F = 128
BLOCK = 256
NEG = -1e9
SLOPE = 0.2


def _prep_kernel(x_ref, w_ref, b_ref, a12_ref, h_ref, f_ref):
    h = jnp.dot(x_ref[...], w_ref[...], preferred_element_type=jnp.float32)
    h = h + b_ref[...]
    h_ref[...] = h
    f_ref[...] = jnp.dot(h, a12_ref[...], preferred_element_type=jnp.float32)


def _attn_kernel(a_ref, at_ref, f1c_ref, f1r_ref, f2r_ref, f2c_ref, h_ref,
                 o_ref):
    h = h_ref[...]

    e1 = f1c_ref[...] + f2r_ref[...]
    e1 = jnp.where(e1 >= 0, e1, SLOPE * e1)
    l1 = jnp.where(a_ref[...] > 0, e1, NEG)
    m1 = jnp.max(l1, axis=1, keepdims=True)
    p1 = jnp.exp(l1 - m1)
    s1 = jnp.sum(p1, axis=1, keepdims=True)
    o1 = jnp.dot(p1, h, preferred_element_type=jnp.float32) / s1

    e2 = f2c_ref[...] + f1r_ref[...]
    e2 = jnp.where(e2 >= 0, e2, SLOPE * e2)
    l2 = jnp.where(at_ref[...] > 0, e2, NEG)
    m2 = jnp.max(l2, axis=0, keepdims=True)
    p2 = jnp.exp(l2 - m2)
    o2 = jax.lax.dot_general(p2, h, (((0,), (0,)), ((), ())),
                             preferred_element_type=jnp.float32)
    ones = jnp.ones((N, 1), dtype=jnp.float32)
    s2 = jax.lax.dot_general(p2, ones, (((0,), (0,)), ((), ())))
    o2 = o2 / s2

    o_ref[...] = 0.5 * (o1 + o2)


@functools.partial(jax.jit, static_argnums=())
def kernel(x, adjacency_matrix, W0, a1_0, a2_0, b0):
    a12 = jnp.concatenate([a1_0, a2_0], axis=1)
    a12 = jnp.pad(a12, ((0, 0), (0, 126)))
    b_row = b0.reshape(1, F)

    bh = 512
    h_full, f_full = pl.pallas_call(
        _prep_kernel,
        grid=(N // bh,),
        in_specs=[
            pl.BlockSpec((bh, F), lambda i: (i, 0)),
            pl.BlockSpec((F, F), lambda i: (0, 0)),
            pl.BlockSpec((1, F), lambda i: (0, 0)),
            pl.BlockSpec((F, 128), lambda i: (0, 0)),
        ],
        out_specs=[
            pl.BlockSpec((bh, F), lambda i: (i, 0)),
            pl.BlockSpec((bh, 128), lambda i: (i, 0)),
        ],
        out_shape=[
            jax.ShapeDtypeStruct((N, F), jnp.float32),
            jax.ShapeDtypeStruct((N, 128), jnp.float32),
        ],
    )(x, W0, b_row, a12)

    f1 = f_full[:, 0:1]
    f2 = f_full[:, 1:2]
    f1_row = f1.reshape(1, N)
    f2_row = f2.reshape(1, N)

    out = pl.pallas_call(
        _attn_kernel,
        grid=(N // BLOCK,),
        in_specs=[
            pl.BlockSpec((BLOCK, N), lambda i: (i, 0)),
            pl.BlockSpec((N, BLOCK), lambda i: (0, i)),
            pl.BlockSpec((BLOCK, 1), lambda i: (i, 0)),
            pl.BlockSpec((1, BLOCK), lambda i: (0, i)),
            pl.BlockSpec((1, N), lambda i: (0, 0)),
            pl.BlockSpec((N, 1), lambda i: (0, 0)),
            pl.BlockSpec((N, F), lambda i: (0, 0)),
        ],
        out_specs=pl.BlockSpec((BLOCK, F), lambda i: (i, 0)),
        out_shape=jax.ShapeDtypeStruct((N, F), jnp.float32),
    )(adjacency_matrix, adjacency_matrix, f1, f1_row, f2_row, f2, h_full)

    return out


# speedup vs baseline: 1.9581x; 1.1463x over previous
import functools

import jax
import jax.numpy as jnp
from jax.experimental import pallas as pl

N = 4096
F = 128
BLOCK = 256
LOG2E = 1.4426950408889634
SLOPE = 0.2


def _prep_kernel(x_ref, w_ref, b_ref, a1_ref, a2_ref,
                 h_ref, u1_ref, u2_ref, v1_ref, v2_ref, hm_ref):
    h = jnp.dot(x_ref[...], w_ref[...], preferred_element_type=jnp.float32)
    h = h + b_ref[...]
    h_ref[...] = h
    f1 = jnp.dot(h, a1_ref[...], preferred_element_type=jnp.float32)
    f2 = jnp.dot(h, a2_ref[...], preferred_element_type=jnp.float32)
    emax = jnp.max(f1) + jnp.max(f2)
    shift = jnp.maximum(emax, SLOPE * emax)
    u1_ref[...] = (f1 - shift) * LOG2E
    u2_ref[...] = (SLOPE * f1 - shift) * LOG2E
    v1_ref[...] = f2 * LOG2E
    v2_ref[...] = f2 * (SLOPE * LOG2E)
    hm_ref[...] = jnp.mean(h, axis=0, keepdims=True)


def _attn_kernel(a_ref, at_ref, u1c_ref, u2c_ref, u1r_ref, u2r_ref,
                 v1r_ref, v2r_ref, v1c_ref, v2c_ref, h_ref, hm_ref,
                 o_ref):
    h = h_ref[...]
    hmean = hm_ref[...]

    arg1 = jnp.maximum(u1c_ref[...] + v1r_ref[...],
                       u2c_ref[...] + v2r_ref[...])
    p1 = a_ref[...] * jnp.exp2(arg1)
    s1 = jnp.sum(p1, axis=1, keepdims=True)
    o1 = jnp.dot(p1, h, preferred_element_type=jnp.float32)
    good1 = s1 > 0
    o1 = jnp.where(good1, o1 / jnp.where(good1, s1, 1.0), hmean)

    arg2 = jnp.maximum(v1c_ref[...] + u1r_ref[...],
                       v2c_ref[...] + u2r_ref[...])
    p2 = at_ref[...] * jnp.exp2(arg2)
    o2 = jax.lax.dot_general(p2, h, (((0,), (0,)), ((), ())),
                             preferred_element_type=jnp.float32)
    ones = jnp.ones((N, 1), dtype=jnp.float32)
    s2 = jax.lax.dot_general(p2, ones, (((0,), (0,)), ((), ())))
    good2 = s2 > 0
    o2 = jnp.where(good2, o2 / jnp.where(good2, s2, 1.0), hmean)

    o_ref[...] = 0.5 * (o1 + o2)


@functools.partial(jax.jit, static_argnums=())
def kernel(x, adjacency_matrix, W0, a1_0, a2_0, b0):
    b_row = b0.reshape(1, F)

    vec = jax.ShapeDtypeStruct((N, 1), jnp.float32)
    h_full, u1, u2, v1, v2, hmean = pl.pallas_call(
        _prep_kernel,
        grid=(1,),
        in_specs=[
            pl.BlockSpec((N, F), lambda i: (0, 0)),
            pl.BlockSpec((F, F), lambda i: (0, 0)),
            pl.BlockSpec((1, F), lambda i: (0, 0)),
            pl.BlockSpec((F, 1), lambda i: (0, 0)),
            pl.BlockSpec((F, 1), lambda i: (0, 0)),
        ],
        out_specs=[
            pl.BlockSpec((N, F), lambda i: (0, 0)),
            pl.BlockSpec((N, 1), lambda i: (0, 0)),
            pl.BlockSpec((N, 1), lambda i: (0, 0)),
            pl.BlockSpec((N, 1), lambda i: (0, 0)),
            pl.BlockSpec((N, 1), lambda i: (0, 0)),
            pl.BlockSpec((1, F), lambda i: (0, 0)),
        ],
        out_shape=[
            jax.ShapeDtypeStruct((N, F), jnp.float32),
            vec, vec, vec, vec,
            jax.ShapeDtypeStruct((1, F), jnp.float32),
        ],
    )(x, W0, b_row, a1_0, a2_0)

    u1r = u1.reshape(1, N)
    u2r = u2.reshape(1, N)
    v1r = v1.reshape(1, N)
    v2r = v2.reshape(1, N)

    out = pl.pallas_call(
        _attn_kernel,
        grid=(N // BLOCK,),
        in_specs=[
            pl.BlockSpec((BLOCK, N), lambda i: (i, 0)),
            pl.BlockSpec((N, BLOCK), lambda i: (0, i)),
            pl.BlockSpec((BLOCK, 1), lambda i: (i, 0)),
            pl.BlockSpec((BLOCK, 1), lambda i: (i, 0)),
            pl.BlockSpec((1, BLOCK), lambda i: (0, i)),
            pl.BlockSpec((1, BLOCK), lambda i: (0, i)),
            pl.BlockSpec((1, N), lambda i: (0, 0)),
            pl.BlockSpec((1, N), lambda i: (0, 0)),
            pl.BlockSpec((N, 1), lambda i: (0, 0)),
            pl.BlockSpec((N, 1), lambda i: (0, 0)),
            pl.BlockSpec((N, F), lambda i: (0, 0)),
            pl.BlockSpec((1, F), lambda i: (0, 0)),
        ],
        out_specs=pl.BlockSpec((BLOCK, F), lambda i: (i, 0)),
        out_shape=jax.ShapeDtypeStruct((N, F), jnp.float32),
    )(adjacency_matrix, adjacency_matrix, u1, u2, u1r, u2r,
      v1r, v2r, v1, v2, h_full, hmean)

    return out


# speedup vs baseline: 1.9589x; 1.0004x over previous
import functools

import jax
import jax.numpy as jnp
from jax.experimental import pallas as pl
from jax.experimental.pallas import tpu as pltpu

N = 4096
F = 128
BLOCK = 256
LOG2E = 1.4426950408889634
SLOPE = 0.2


def _prep_kernel(x_ref, w_ref, b_ref, a1_ref, a2_ref,
                 h_ref, u1_ref, u2_ref, v1_ref, v2_ref, hm_ref):
    h = jnp.dot(x_ref[...], w_ref[...], preferred_element_type=jnp.float32)
    h = h + b_ref[...]
    h_ref[...] = h
    f1 = jnp.dot(h, a1_ref[...], preferred_element_type=jnp.float32)
    f2 = jnp.dot(h, a2_ref[...], preferred_element_type=jnp.float32)
    emax = jnp.max(f1) + jnp.max(f2)
    shift = jnp.maximum(emax, SLOPE * emax)
    u1_ref[...] = (f1 - shift) * LOG2E
    u2_ref[...] = (SLOPE * f1 - shift) * LOG2E
    v1_ref[...] = f2 * LOG2E
    v2_ref[...] = f2 * (SLOPE * LOG2E)
    hm_ref[...] = jnp.mean(h, axis=0, keepdims=True)


def _attn_kernel(a_ref, at_ref, u1c_ref, u2c_ref, u1r_ref, u2r_ref,
                 v1r_ref, v2r_ref, v1c_ref, v2c_ref, h_ref, hm_ref,
                 o_ref):
    h = h_ref[...]
    hmean = hm_ref[...]

    arg1 = jnp.maximum(u1c_ref[...] + v1r_ref[...],
                       u2c_ref[...] + v2r_ref[...])
    p1 = a_ref[...] * jnp.exp2(arg1)
    s1 = jnp.sum(p1, axis=1, keepdims=True)
    o1 = jnp.dot(p1, h, preferred_element_type=jnp.float32)
    good1 = s1 > 0
    o1 = jnp.where(good1, o1 / jnp.where(good1, s1, 1.0), hmean)

    arg2 = jnp.maximum(v1c_ref[...] + u1r_ref[...],
                       v2c_ref[...] + u2r_ref[...])
    p2 = at_ref[...] * jnp.exp2(arg2)
    o2 = jax.lax.dot_general(p2, h, (((0,), (0,)), ((), ())),
                             preferred_element_type=jnp.float32)
    ones = jnp.ones((N, 1), dtype=jnp.float32)
    s2 = jax.lax.dot_general(p2, ones, (((0,), (0,)), ((), ())))
    good2 = s2 > 0
    o2 = jnp.where(good2, o2 / jnp.where(good2, s2, 1.0), hmean)

    o_ref[...] = 0.5 * (o1 + o2)


@functools.partial(jax.jit, static_argnums=())
def kernel(x, adjacency_matrix, W0, a1_0, a2_0, b0):
    b_row = b0.reshape(1, F)

    vec = jax.ShapeDtypeStruct((N, 1), jnp.float32)
    h_full, u1, u2, v1, v2, hmean = pl.pallas_call(
        _prep_kernel,
        grid=(1,),
        in_specs=[
            pl.BlockSpec((N, F), lambda i: (0, 0)),
            pl.BlockSpec((F, F), lambda i: (0, 0)),
            pl.BlockSpec((1, F), lambda i: (0, 0)),
            pl.BlockSpec((F, 1), lambda i: (0, 0)),
            pl.BlockSpec((F, 1), lambda i: (0, 0)),
        ],
        out_specs=[
            pl.BlockSpec((N, F), lambda i: (0, 0)),
            pl.BlockSpec((N, 1), lambda i: (0, 0)),
            pl.BlockSpec((N, 1), lambda i: (0, 0)),
            pl.BlockSpec((N, 1), lambda i: (0, 0)),
            pl.BlockSpec((N, 1), lambda i: (0, 0)),
            pl.BlockSpec((1, F), lambda i: (0, 0)),
        ],
        out_shape=[
            jax.ShapeDtypeStruct((N, F), jnp.float32),
            vec, vec, vec, vec,
            jax.ShapeDtypeStruct((1, F), jnp.float32),
        ],
    )(x, W0, b_row, a1_0, a2_0)

    u1r = u1.reshape(1, N)
    u2r = u2.reshape(1, N)
    v1r = v1.reshape(1, N)
    v2r = v2.reshape(1, N)

    out = pl.pallas_call(
        _attn_kernel,
        grid=(N // BLOCK,),
        in_specs=[
            pl.BlockSpec((BLOCK, N), lambda i: (i, 0)),
            pl.BlockSpec((N, BLOCK), lambda i: (0, i)),
            pl.BlockSpec((BLOCK, 1), lambda i: (i, 0)),
            pl.BlockSpec((BLOCK, 1), lambda i: (i, 0)),
            pl.BlockSpec((1, BLOCK), lambda i: (0, i)),
            pl.BlockSpec((1, BLOCK), lambda i: (0, i)),
            pl.BlockSpec((1, N), lambda i: (0, 0)),
            pl.BlockSpec((1, N), lambda i: (0, 0)),
            pl.BlockSpec((N, 1), lambda i: (0, 0)),
            pl.BlockSpec((N, 1), lambda i: (0, 0)),
            pl.BlockSpec((N, F), lambda i: (0, 0)),
            pl.BlockSpec((1, F), lambda i: (0, 0)),
        ],
        out_specs=pl.BlockSpec((BLOCK, F), lambda i: (i, 0)),
        out_shape=jax.ShapeDtypeStruct((N, F), jnp.float32),
        compiler_params=pltpu.CompilerParams(
            dimension_semantics=("parallel",)),
    )(adjacency_matrix, adjacency_matrix, u1, u2, u1r, u2r,
      v1r, v2r, v1, v2, h_full, hmean)

    return out


# speedup vs baseline: 2.2842x; 1.1661x over previous
import functools

import jax
import jax.numpy as jnp
from jax.experimental import pallas as pl
from jax.experimental.pallas import tpu as pltpu

N = 4096
F = 128
BLOCK = 256
NBLK = N // BLOCK
LOG2E = 1.4426950408889634
SLOPE = 0.2


def _prep_kernel(x_ref, w_ref, b_ref, a1_ref, a2_ref,
                 h_ref, u1_ref, u2_ref, v1_ref, v2_ref, hm_ref):
    h = jnp.dot(x_ref[...], w_ref[...], preferred_element_type=jnp.float32)
    h = h + b_ref[...]
    h_ref[...] = h
    f1 = jnp.dot(h, a1_ref[...], preferred_element_type=jnp.float32)
    f2 = jnp.dot(h, a2_ref[...], preferred_element_type=jnp.float32)
    emax = jnp.max(f1) + jnp.max(f2)
    shift = jnp.maximum(emax, SLOPE * emax)
    u1_ref[...] = (f1 - shift) * LOG2E
    u2_ref[...] = (SLOPE * f1 - shift) * LOG2E
    v1_ref[...] = f2 * LOG2E
    v2_ref[...] = f2 * (SLOPE * LOG2E)
    hm_ref[...] = jnp.mean(h, axis=0, keepdims=True)


def _attn_kernel(a_ref, u1c_ref, u2c_ref, v1c_ref, v2c_ref,
                 u1r_ref, u2r_ref, v1r_ref, v2r_ref, h_ref, hm_ref,
                 o_ref, acc2_ref, s2_ref):
    i = pl.program_id(0)

    @pl.when(i == 0)
    def _init():
        acc2_ref[...] = jnp.zeros_like(acc2_ref)
        s2_ref[...] = jnp.zeros_like(s2_ref)

    a = a_ref[...]
    h = h_ref[...]
    hmean = hm_ref[...]

    arg1 = jnp.maximum(u1c_ref[...] + v1r_ref[...],
                       u2c_ref[...] + v2r_ref[...])
    p1 = a * jnp.exp2(arg1)
    s1 = jnp.sum(p1, axis=1, keepdims=True)
    o1 = jnp.dot(p1, h, preferred_element_type=jnp.float32)
    good1 = s1 > 0
    o1 = jnp.where(good1, o1 / jnp.where(good1, s1, 1.0), hmean)
    o_ref[pl.ds(i * BLOCK, BLOCK), :] = o1

    arg2 = jnp.maximum(u1r_ref[...] + v1c_ref[...],
                       u2r_ref[...] + v2c_ref[...])
    p2 = a * jnp.exp2(arg2)
    hb = h_ref[pl.ds(i * BLOCK, BLOCK), :]
    acc2_ref[...] += jax.lax.dot_general(
        p2, hb, (((0,), (0,)), ((), ())),
        preferred_element_type=jnp.float32)
    ones = jnp.ones((BLOCK, 1), dtype=jnp.float32)
    s2_ref[...] += jax.lax.dot_general(
        p2, ones, (((0,), (0,)), ((), ())))

    @pl.when(i == NBLK - 1)
    def _finish():
        s2 = s2_ref[...]
        good2 = s2 > 0
        o2 = jnp.where(good2,
                       acc2_ref[...] / jnp.where(good2, s2, 1.0), hmean)
        o_ref[...] = 0.5 * (o_ref[...] + o2)


@functools.partial(jax.jit, static_argnums=())
def kernel(x, adjacency_matrix, W0, a1_0, a2_0, b0):
    b_row = b0.reshape(1, F)

    vec = jax.ShapeDtypeStruct((N, 1), jnp.float32)
    h_full, u1, u2, v1, v2, hmean = pl.pallas_call(
        _prep_kernel,
        grid=(1,),
        in_specs=[
            pl.BlockSpec((N, F), lambda i: (0, 0)),
            pl.BlockSpec((F, F), lambda i: (0, 0)),
            pl.BlockSpec((1, F), lambda i: (0, 0)),
            pl.BlockSpec((F, 1), lambda i: (0, 0)),
            pl.BlockSpec((F, 1), lambda i: (0, 0)),
        ],
        out_specs=[
            pl.BlockSpec((N, F), lambda i: (0, 0)),
            pl.BlockSpec((N, 1), lambda i: (0, 0)),
            pl.BlockSpec((N, 1), lambda i: (0, 0)),
            pl.BlockSpec((N, 1), lambda i: (0, 0)),
            pl.BlockSpec((N, 1), lambda i: (0, 0)),
            pl.BlockSpec((1, F), lambda i: (0, 0)),
        ],
        out_shape=[
            jax.ShapeDtypeStruct((N, F), jnp.float32),
            vec, vec, vec, vec,
            jax.ShapeDtypeStruct((1, F), jnp.float32),
        ],
    )(x, W0, b_row, a1_0, a2_0)

    u1r = u1.reshape(1, N)
    u2r = u2.reshape(1, N)
    v1r = v1.reshape(1, N)
    v2r = v2.reshape(1, N)

    out = pl.pallas_call(
        _attn_kernel,
        grid=(NBLK,),
        in_specs=[
            pl.BlockSpec((BLOCK, N), lambda i: (i, 0)),
            pl.BlockSpec((BLOCK, 1), lambda i: (i, 0)),
            pl.BlockSpec((BLOCK, 1), lambda i: (i, 0)),
            pl.BlockSpec((BLOCK, 1), lambda i: (i, 0)),
            pl.BlockSpec((BLOCK, 1), lambda i: (i, 0)),
            pl.BlockSpec((1, N), lambda i: (0, 0)),
            pl.BlockSpec((1, N), lambda i: (0, 0)),
            pl.BlockSpec((1, N), lambda i: (0, 0)),
            pl.BlockSpec((1, N), lambda i: (0, 0)),
            pl.BlockSpec((N, F), lambda i: (0, 0)),
            pl.BlockSpec((1, F), lambda i: (0, 0)),
        ],
        out_specs=pl.BlockSpec((N, F), lambda i: (0, 0)),
        out_shape=jax.ShapeDtypeStruct((N, F), jnp.float32),
        scratch_shapes=[
            pltpu.VMEM((N, F), jnp.float32),
            pltpu.VMEM((N, 1), jnp.float32),
        ],
    )(adjacency_matrix, u1, u2, v1, v2,
      u1r, u2r, v1r, v2r, h_full, hmean)

    return out


# speedup vs baseline: 2.3481x; 1.0280x over previous
import functools

import jax
import jax.numpy as jnp
from jax.experimental import pallas as pl
from jax.experimental.pallas import tpu as pltpu

N = 4096
F = 128
BLOCK = 256
NBLK = N // BLOCK
LOG2E = 1.4426950408889634
SLOPE = 0.2


def _prep_kernel(x_ref, w_ref, b_ref, a1_ref, a2_ref,
                 h_ref, u1_ref, u2_ref, v1_ref, v2_ref, hm_ref):
    h = jnp.dot(x_ref[...], w_ref[...], preferred_element_type=jnp.float32)
    h = h + b_ref[...]
    h_ref[...] = h
    f1 = jnp.dot(h, a1_ref[...], preferred_element_type=jnp.float32)
    f2 = jnp.dot(h, a2_ref[...], preferred_element_type=jnp.float32)
    emax = jnp.max(f1) + jnp.max(f2)
    shift = jnp.maximum(emax, SLOPE * emax)
    u1_ref[...] = (f1 - shift) * LOG2E
    u2_ref[...] = (SLOPE * f1 - shift) * LOG2E
    v1_ref[...] = f2 * LOG2E
    v2_ref[...] = f2 * (SLOPE * LOG2E)
    hm_ref[...] = jnp.mean(h, axis=0, keepdims=True)


def _attn_kernel(a_ref, u1c_ref, u2c_ref, v1c_ref, v2c_ref,
                 u1r_ref, u2r_ref, v1r_ref, v2r_ref, h_ref, hm_ref,
                 o_ref, acc2_ref, s2_ref):
    i = pl.program_id(0)

    @pl.when(i == 0)
    def _init():
        acc2_ref[...] = jnp.zeros_like(acc2_ref)
        s2_ref[...] = jnp.zeros_like(s2_ref)

    a = a_ref[...]
    h = h_ref[...]
    hmean = hm_ref[...]

    arg1 = jnp.maximum(u1c_ref[...] + v1r_ref[...],
                       u2c_ref[...] + v2r_ref[...])
    p1 = a * jnp.exp2(arg1)
    ones_n = jnp.ones((N, 1), dtype=jnp.float32)
    s1 = jnp.dot(p1, ones_n, preferred_element_type=jnp.float32)
    o1 = jnp.dot(p1, h, preferred_element_type=jnp.float32)
    good1 = s1 > 0
    o1 = jnp.where(good1, o1 / jnp.where(good1, s1, 1.0), hmean)
    o_ref[pl.ds(i * BLOCK, BLOCK), :] = o1

    arg2 = jnp.maximum(u1r_ref[...] + v1c_ref[...],
                       u2r_ref[...] + v2c_ref[...])
    p2 = a * jnp.exp2(arg2)
    hb = h_ref[pl.ds(i * BLOCK, BLOCK), :]
    acc2_ref[...] += jax.lax.dot_general(
        p2, hb, (((0,), (0,)), ((), ())),
        preferred_element_type=jnp.float32)
    ones = jnp.ones((BLOCK, 1), dtype=jnp.float32)
    s2_ref[...] += jax.lax.dot_general(
        p2, ones, (((0,), (0,)), ((), ())))

    @pl.when(i == NBLK - 1)
    def _finish():
        s2 = s2_ref[...]
        good2 = s2 > 0
        o2 = jnp.where(good2,
                       acc2_ref[...] / jnp.where(good2, s2, 1.0), hmean)
        o_ref[...] = 0.5 * (o_ref[...] + o2)


@functools.partial(jax.jit, static_argnums=())
def kernel(x, adjacency_matrix, W0, a1_0, a2_0, b0):
    b_row = b0.reshape(1, F)

    vec = jax.ShapeDtypeStruct((N, 1), jnp.float32)
    h_full, u1, u2, v1, v2, hmean = pl.pallas_call(
        _prep_kernel,
        grid=(1,),
        in_specs=[
            pl.BlockSpec((N, F), lambda i: (0, 0)),
            pl.BlockSpec((F, F), lambda i: (0, 0)),
            pl.BlockSpec((1, F), lambda i: (0, 0)),
            pl.BlockSpec((F, 1), lambda i: (0, 0)),
            pl.BlockSpec((F, 1), lambda i: (0, 0)),
        ],
        out_specs=[
            pl.BlockSpec((N, F), lambda i: (0, 0)),
            pl.BlockSpec((N, 1), lambda i: (0, 0)),
            pl.BlockSpec((N, 1), lambda i: (0, 0)),
            pl.BlockSpec((N, 1), lambda i: (0, 0)),
            pl.BlockSpec((N, 1), lambda i: (0, 0)),
            pl.BlockSpec((1, F), lambda i: (0, 0)),
        ],
        out_shape=[
            jax.ShapeDtypeStruct((N, F), jnp.float32),
            vec, vec, vec, vec,
            jax.ShapeDtypeStruct((1, F), jnp.float32),
        ],
    )(x, W0, b_row, a1_0, a2_0)

    u1r = u1.reshape(1, N)
    u2r = u2.reshape(1, N)
    v1r = v1.reshape(1, N)
    v2r = v2.reshape(1, N)

    out = pl.pallas_call(
        _attn_kernel,
        grid=(NBLK,),
        in_specs=[
            pl.BlockSpec((BLOCK, N), lambda i: (i, 0)),
            pl.BlockSpec((BLOCK, 1), lambda i: (i, 0)),
            pl.BlockSpec((BLOCK, 1), lambda i: (i, 0)),
            pl.BlockSpec((BLOCK, 1), lambda i: (i, 0)),
            pl.BlockSpec((BLOCK, 1), lambda i: (i, 0)),
            pl.BlockSpec((1, N), lambda i: (0, 0)),
            pl.BlockSpec((1, N), lambda i: (0, 0)),
            pl.BlockSpec((1, N), lambda i: (0, 0)),
            pl.BlockSpec((1, N), lambda i: (0, 0)),
            pl.BlockSpec((N, F), lambda i: (0, 0)),
            pl.BlockSpec((1, F), lambda i: (0, 0)),
        ],
        out_specs=pl.BlockSpec((N, F), lambda i: (0, 0)),
        out_shape=jax.ShapeDtypeStruct((N, F), jnp.float32),
        scratch_shapes=[
            pltpu.VMEM((N, F), jnp.float32),
            pltpu.VMEM((N, 1), jnp.float32),
        ],
    )(adjacency_matrix, u1, u2, v1, v2,
      u1r, u2r, v1r, v2r, h_full, hmean)

    return out


# speedup vs baseline: 2.3723x; 1.0103x over previous
import functools

import jax
import jax.numpy as jnp
from jax.experimental import pallas as pl
from jax.experimental.pallas import tpu as pltpu

N = 4096
F = 128
BLOCK = 512
NBLK = N // BLOCK
LOG2E = 1.4426950408889634
SLOPE = 0.2


def _prep_kernel(x_ref, w_ref, b_ref, a1_ref, a2_ref,
                 h_ref, u1_ref, u2_ref, v1_ref, v2_ref, hm_ref):
    h = jnp.dot(x_ref[...], w_ref[...], preferred_element_type=jnp.float32)
    h = h + b_ref[...]
    h_ref[...] = h
    f1 = jnp.dot(h, a1_ref[...], preferred_element_type=jnp.float32)
    f2 = jnp.dot(h, a2_ref[...], preferred_element_type=jnp.float32)
    emax = jnp.max(f1) + jnp.max(f2)
    shift = jnp.maximum(emax, SLOPE * emax)
    u1_ref[...] = (f1 - shift) * LOG2E
    u2_ref[...] = (SLOPE * f1 - shift) * LOG2E
    v1_ref[...] = f2 * LOG2E
    v2_ref[...] = f2 * (SLOPE * LOG2E)
    hm_ref[...] = jnp.mean(h, axis=0, keepdims=True)


def _attn_kernel(a_ref, u1c_ref, u2c_ref, v1c_ref, v2c_ref,
                 u1r_ref, u2r_ref, v1r_ref, v2r_ref, h_ref, hm_ref,
                 o_ref, acc2_ref, s2_ref):
    i = pl.program_id(0)

    @pl.when(i == 0)
    def _init():
        acc2_ref[...] = jnp.zeros_like(acc2_ref)
        s2_ref[...] = jnp.zeros_like(s2_ref)

    a = a_ref[...]
    h = h_ref[...]
    hmean = hm_ref[...]

    arg1 = jnp.maximum(u1c_ref[...] + v1r_ref[...],
                       u2c_ref[...] + v2r_ref[...])
    p1 = a * jnp.exp2(arg1)
    ones_n = jnp.ones((N, 1), dtype=jnp.float32)
    s1 = jnp.dot(p1, ones_n, preferred_element_type=jnp.float32)
    o1 = jnp.dot(p1, h, preferred_element_type=jnp.float32)
    good1 = s1 > 0
    o1 = jnp.where(good1, o1 / jnp.where(good1, s1, 1.0), hmean)
    o_ref[pl.ds(i * BLOCK, BLOCK), :] = o1

    arg2 = jnp.maximum(u1r_ref[...] + v1c_ref[...],
                       u2r_ref[...] + v2c_ref[...])
    p2 = a * jnp.exp2(arg2)
    hb = h_ref[pl.ds(i * BLOCK, BLOCK), :]
    acc2_ref[...] += jax.lax.dot_general(
        p2, hb, (((0,), (0,)), ((), ())),
        preferred_element_type=jnp.float32)
    ones = jnp.ones((BLOCK, 1), dtype=jnp.float32)
    s2_ref[...] += jax.lax.dot_general(
        p2, ones, (((0,), (0,)), ((), ())))

    @pl.when(i == NBLK - 1)
    def _finish():
        s2 = s2_ref[...]
        good2 = s2 > 0
        o2 = jnp.where(good2,
                       acc2_ref[...] / jnp.where(good2, s2, 1.0), hmean)
        o_ref[...] = 0.5 * (o_ref[...] + o2)


@functools.partial(jax.jit, static_argnums=())
def kernel(x, adjacency_matrix, W0, a1_0, a2_0, b0):
    b_row = b0.reshape(1, F)

    vec = jax.ShapeDtypeStruct((N, 1), jnp.float32)
    h_full, u1, u2, v1, v2, hmean = pl.pallas_call(
        _prep_kernel,
        grid=(1,),
        in_specs=[
            pl.BlockSpec((N, F), lambda i: (0, 0)),
            pl.BlockSpec((F, F), lambda i: (0, 0)),
            pl.BlockSpec((1, F), lambda i: (0, 0)),
            pl.BlockSpec((F, 1), lambda i: (0, 0)),
            pl.BlockSpec((F, 1), lambda i: (0, 0)),
        ],
        out_specs=[
            pl.BlockSpec((N, F), lambda i: (0, 0)),
            pl.BlockSpec((N, 1), lambda i: (0, 0)),
            pl.BlockSpec((N, 1), lambda i: (0, 0)),
            pl.BlockSpec((N, 1), lambda i: (0, 0)),
            pl.BlockSpec((N, 1), lambda i: (0, 0)),
            pl.BlockSpec((1, F), lambda i: (0, 0)),
        ],
        out_shape=[
            jax.ShapeDtypeStruct((N, F), jnp.float32),
            vec, vec, vec, vec,
            jax.ShapeDtypeStruct((1, F), jnp.float32),
        ],
    )(x, W0, b_row, a1_0, a2_0)

    u1r = u1.reshape(1, N)
    u2r = u2.reshape(1, N)
    v1r = v1.reshape(1, N)
    v2r = v2.reshape(1, N)

    out = pl.pallas_call(
        _attn_kernel,
        grid=(NBLK,),
        in_specs=[
            pl.BlockSpec((BLOCK, N), lambda i: (i, 0)),
            pl.BlockSpec((BLOCK, 1), lambda i: (i, 0)),
            pl.BlockSpec((BLOCK, 1), lambda i: (i, 0)),
            pl.BlockSpec((BLOCK, 1), lambda i: (i, 0)),
            pl.BlockSpec((BLOCK, 1), lambda i: (i, 0)),
            pl.BlockSpec((1, N), lambda i: (0, 0)),
            pl.BlockSpec((1, N), lambda i: (0, 0)),
            pl.BlockSpec((1, N), lambda i: (0, 0)),
            pl.BlockSpec((1, N), lambda i: (0, 0)),
            pl.BlockSpec((N, F), lambda i: (0, 0)),
            pl.BlockSpec((1, F), lambda i: (0, 0)),
        ],
        out_specs=pl.BlockSpec((N, F), lambda i: (0, 0)),
        out_shape=jax.ShapeDtypeStruct((N, F), jnp.float32),
        scratch_shapes=[
            pltpu.VMEM((N, F), jnp.float32),
            pltpu.VMEM((N, 1), jnp.float32),
        ],
    )(adjacency_matrix, u1, u2, v1, v2,
      u1r, u2r, v1r, v2r, h_full, hmean)

    return out


# speedup vs baseline: 2.5230x; 1.0635x over previous
import functools

import jax
import jax.numpy as jnp
from jax.experimental import pallas as pl
from jax.experimental.pallas import tpu as pltpu

N = 4096
F = 128
BLOCK = 512
NBLK = N // BLOCK
LOG2E = 1.4426950408889634
SLOPE = 0.2


def _prep_kernel(x_ref, w_ref, b_ref, a1_ref, a2_ref,
                 h_ref, u1_ref, u2_ref, v1_ref, v2_ref, hm_ref):
    h = jnp.dot(x_ref[...], w_ref[...], preferred_element_type=jnp.float32)
    h = h + b_ref[...]
    h_ref[...] = h
    f1 = jnp.dot(h, a1_ref[...], preferred_element_type=jnp.float32)
    f2 = jnp.dot(h, a2_ref[...], preferred_element_type=jnp.float32)
    emax = jnp.max(f1) + jnp.max(f2)
    shift = jnp.maximum(emax, SLOPE * emax)
    u1_ref[...] = (f1 - shift) * LOG2E
    u2_ref[...] = (SLOPE * f1 - shift) * LOG2E
    v1_ref[...] = f2 * LOG2E
    v2_ref[...] = f2 * (SLOPE * LOG2E)
    hm_ref[...] = jnp.mean(h, axis=0, keepdims=True)


def _attn_kernel(a_ref, u1c_ref, u2c_ref, v1c_ref, v2c_ref,
                 u1r_ref, u2r_ref, v1r_ref, v2r_ref, h_ref, hm_ref,
                 o_ref, acc2_ref, s2_ref):
    i = pl.program_id(0)

    @pl.when(i == 0)
    def _init():
        acc2_ref[...] = jnp.zeros_like(acc2_ref)
        s2_ref[...] = jnp.zeros_like(s2_ref)

    a = a_ref[...]
    h = h_ref[...]
    hmean = hm_ref[...]

    arg1 = jnp.maximum(u1c_ref[...] + v1r_ref[...],
                       u2c_ref[...] + v2r_ref[...])
    p1 = a * jnp.exp2(arg1)
    ones_n = jnp.ones((N, 1), dtype=jnp.float32)
    s1 = jnp.dot(p1, ones_n, preferred_element_type=jnp.float32)
    o1 = jnp.dot(p1, h, preferred_element_type=jnp.float32)
    good1 = s1 > 0
    o1 = jnp.where(good1, o1 / jnp.where(good1, s1, 1.0), hmean)
    o_ref[pl.ds(i * BLOCK, BLOCK), :] = o1

    arg2 = jnp.maximum(u1r_ref[...] + v1c_ref[...],
                       u2r_ref[...] + v2c_ref[...])
    p2 = a * jnp.exp2(arg2)
    hb = h_ref[pl.ds(i * BLOCK, BLOCK), :]
    acc2_ref[...] += jax.lax.dot_general(
        p2, hb, (((0,), (0,)), ((), ())),
        preferred_element_type=jnp.float32)
    ones_row = jnp.ones((1, BLOCK), dtype=jnp.float32)
    s2_ref[...] += jnp.dot(ones_row, p2,
                           preferred_element_type=jnp.float32)

    @pl.when(i == NBLK - 1)
    def _finish():
        s2 = jnp.transpose(s2_ref[...], (1, 0))
        good2 = s2 > 0
        o2 = jnp.where(good2,
                       acc2_ref[...] / jnp.where(good2, s2, 1.0), hmean)
        o_ref[...] = 0.5 * (o_ref[...] + o2)


@functools.partial(jax.jit, static_argnums=())
def kernel(x, adjacency_matrix, W0, a1_0, a2_0, b0):
    b_row = b0.reshape(1, F)

    vec = jax.ShapeDtypeStruct((N, 1), jnp.float32)
    h_full, u1, u2, v1, v2, hmean = pl.pallas_call(
        _prep_kernel,
        grid=(1,),
        in_specs=[
            pl.BlockSpec((N, F), lambda i: (0, 0)),
            pl.BlockSpec((F, F), lambda i: (0, 0)),
            pl.BlockSpec((1, F), lambda i: (0, 0)),
            pl.BlockSpec((F, 1), lambda i: (0, 0)),
            pl.BlockSpec((F, 1), lambda i: (0, 0)),
        ],
        out_specs=[
            pl.BlockSpec((N, F), lambda i: (0, 0)),
            pl.BlockSpec((N, 1), lambda i: (0, 0)),
            pl.BlockSpec((N, 1), lambda i: (0, 0)),
            pl.BlockSpec((N, 1), lambda i: (0, 0)),
            pl.BlockSpec((N, 1), lambda i: (0, 0)),
            pl.BlockSpec((1, F), lambda i: (0, 0)),
        ],
        out_shape=[
            jax.ShapeDtypeStruct((N, F), jnp.float32),
            vec, vec, vec, vec,
            jax.ShapeDtypeStruct((1, F), jnp.float32),
        ],
    )(x, W0, b_row, a1_0, a2_0)

    u1r = u1.reshape(1, N)
    u2r = u2.reshape(1, N)
    v1r = v1.reshape(1, N)
    v2r = v2.reshape(1, N)

    out = pl.pallas_call(
        _attn_kernel,
        grid=(NBLK,),
        in_specs=[
            pl.BlockSpec((BLOCK, N), lambda i: (i, 0)),
            pl.BlockSpec((BLOCK, 1), lambda i: (i, 0)),
            pl.BlockSpec((BLOCK, 1), lambda i: (i, 0)),
            pl.BlockSpec((BLOCK, 1), lambda i: (i, 0)),
            pl.BlockSpec((BLOCK, 1), lambda i: (i, 0)),
            pl.BlockSpec((1, N), lambda i: (0, 0)),
            pl.BlockSpec((1, N), lambda i: (0, 0)),
            pl.BlockSpec((1, N), lambda i: (0, 0)),
            pl.BlockSpec((1, N), lambda i: (0, 0)),
            pl.BlockSpec((N, F), lambda i: (0, 0)),
            pl.BlockSpec((1, F), lambda i: (0, 0)),
        ],
        out_specs=pl.BlockSpec((N, F), lambda i: (0, 0)),
        out_shape=jax.ShapeDtypeStruct((N, F), jnp.float32),
        scratch_shapes=[
            pltpu.VMEM((N, F), jnp.float32),
            pltpu.VMEM((1, N), jnp.float32),
        ],
    )(adjacency_matrix, u1, u2, v1, v2,
      u1r, u2r, v1r, v2r, h_full, hmean)

    return out
